# baseline (speedup 1.0000x reference)
import functools

import jax
import jax.numpy as jnp
import numpy as np
from jax import lax
from jax.experimental import pallas as pl
from jax.experimental.pallas import tpu as pltpu
from jax.experimental.pallas import tpu_sc as plsc

KNN = 32
L = 16
NC, NS = 2, 16
NW = NC * NS


def _scores_kernel(xb, xm1, xm2, cw, cb, Wq, bq, keys1, keys2,
                   s1_out, s2_out, *, heads, key_dim, key_num):
    half = key_dim // 2
    conv = (cb[0][None, :]
            + xm2[...] * cw[0][None, :]
            + xm1[...] * cw[1][None, :]
            + xb[...] * cw[2][None, :])
    query = lax.dot_general(conv, Wq[...], (((1,), (1,)), ((), ())),
                            preferred_element_type=jnp.float32)
    query = query + bq[0][None, :]
    for h in range(heads):
        q1 = query[:, h * key_dim: h * key_dim + half]
        q2 = query[:, h * key_dim + half: (h + 1) * key_dim]
        k1 = keys1[h * key_num:(h + 1) * key_num, :]
        k2 = keys2[h * key_num:(h + 1) * key_num, :]
        s1_out[h, :, :] = lax.dot_general(
            q1, k1, (((1,), (1,)), ((), ())),
            preferred_element_type=jnp.float32)
        s2_out[h, :, :] = lax.dot_general(
            q2, k2, (((1,), (1,)), ((), ())),
            preferred_element_type=jnp.float32)


def _cand_table():
    pairs = [(r, c) for r in range(KNN) for c in range(KNN)
             if (r + 1) * (c + 1) <= KNN]
    pairs.sort(key=lambda rc: rc[0] * KNN + rc[1])
    npad = 128
    tab = np.zeros((4, npad), np.int32)
    tab[2, :] = 4 * KNN * KNN
    for j, (r, c) in enumerate(pairs):
        tab[0, j] = r
        tab[1, j] = c
        tab[2, j] = r * KNN + c
    return tab


_SC_CAND = _cand_table()


def _cmpsel(a, ia, b, ib):
    m = a >= b
    return (jnp.where(m, a, b), jnp.where(m, ia, ib),
            jnp.where(m, b, a), jnp.where(m, ib, ia))


def _sortkv(k, v, descending):
    return plsc.sort_key_val(k, v, descending=descending)


def _pair_init(c1, i1, c2, i2):
    b1k, b1v = _sortkv(c1, i1, False)
    b2k, b2v = _sortkv(c2, i2, True)
    hiB, ihiB, loB, iloB = _cmpsel(b1k, b1v, b2k, b2v)
    A0, IA0 = _sortkv(hiB, ihiB, True)
    A1, IA1 = _sortkv(loB, iloB, True)
    return A0, IA0, A1, IA1


def _pair_step(A0, IA0, A1, IA1, c1, i1, c2, i2):
    b1k, b1v = _sortkv(c1, i1, False)
    b2k, b2v = _sortkv(c2, i2, True)
    hiB, ihiB, loB, iloB = _cmpsel(b1k, b1v, b2k, b2v)
    Basc0, iB0 = _sortkv(loB, iloB, False)
    Basc1, iB1 = _sortkv(hiB, ihiB, False)
    h0, ih0, _, _ = _cmpsel(A0, IA0, Basc0, iB0)
    h1, ih1, _, _ = _cmpsel(A1, IA1, Basc1, iB1)
    u, iu, lo, ilo = _cmpsel(h0, ih0, h1, ih1)
    A0, IA0 = _sortkv(u, iu, True)
    A1, IA1 = _sortkv(lo, ilo, True)
    return A0, IA0, A1, IA1


def _pair_init_asc(c1, i1, c2, i2):
    b1k, b1v = _sortkv(c1, i1, False)
    b2k, b2v = _sortkv(c2, i2, True)
    hiB, ihiB, loB, iloB = _cmpsel(b1k, b1v, b2k, b2v)
    B0, iB0 = _sortkv(loB, iloB, False)
    B1, iB1 = _sortkv(hiB, ihiB, False)
    return B0, iB0, B1, iB1


def _merge_A_Basc(A0, IA0, A1, IA1, B0, iB0, B1, iB1):
    h0, ih0, _, _ = _cmpsel(A0, IA0, B0, iB0)
    h1, ih1, _, _ = _cmpsel(A1, IA1, B1, iB1)
    u, iu, lo, ilo = _cmpsel(h0, ih0, h1, ih1)
    A0, IA0 = _sortkv(u, iu, True)
    A1, IA1 = _sortkv(lo, ilo, True)
    return A0, IA0, A1, IA1


_XMASK = 0x7FFFFFFF
_CAP = 48


def _row_top32(buf, par, h, rb, n_chunks, mk, sv, si, slot):
    iota = lax.broadcasted_iota(jnp.int32, (L,), 0)
    zero = jnp.int32(0)
    one = jnp.int32(1)

    for g in range(n_chunks):
        v = buf[par, h, rb, pl.ds(g * L, L)]
        u = plsc.bitcast(v, jnp.int32)
        mk[slot, pl.ds(g * L, L)] = jnp.where(u < zero, u ^ _XMASK, u)

    def count_ge(T):
        acc = jnp.zeros((L,), jnp.int32)
        for g in range(n_chunks):
            sk = mk[slot, pl.ds(g * L, L)]
            acc = acc + jnp.where(sk >= T, 1, 0)
        return jnp.sum(acc)

    def cond(carry):
        Tlo, Thi, clo = carry
        return jnp.logical_and(clo > _CAP, Thi > Tlo + one)

    def body(carry):
        Tlo, Thi, clo = carry
        mid = (Tlo >> one) + (Thi >> one) + (Tlo & Thi & one)
        c = count_ge(mid)
        pred = c >= 32
        return (jnp.where(pred, mid, Tlo), jnp.where(pred, Thi, mid),
                jnp.where(pred, c, clo))

    Tlo, _, _ = lax.while_loop(
        cond, body,
        (jnp.int32(-2**31), jnp.int32(2**31 - 1), jnp.int32(512)))

    neginf = jnp.float32(-jnp.inf)
    for q in range(4):
        sv[slot, pl.ds(q * L, L)] = jnp.full((L,), neginf, jnp.float32)
    off = zero
    for g in range(n_chunks):
        sk = mk[slot, pl.ds(g * L, L)]
        m = sk >= Tlo
        offc = jnp.minimum(off, jnp.int32(_CAP))
        plsc.store_compressed(sv.at[slot, pl.ds(offc, L)],
                              buf[par, h, rb, pl.ds(g * L, L)], mask=m)
        plsc.store_compressed(si.at[slot, pl.ds(offc, L)],
                              iota + g * L, mask=m)
        off = off + jnp.sum(jnp.where(m, 1, 0))

    A = _pair_init(sv[slot, pl.ds(0, L)], si[slot, pl.ds(0, L)],
                   sv[slot, pl.ds(L, L)], si[slot, pl.ds(L, L)])
    B = _pair_init_asc(sv[slot, pl.ds(2 * L, L)], si[slot, pl.ds(2 * L, L)],
                       sv[slot, pl.ds(3 * L, L)], si[slot, pl.ds(3 * L, L)])
    return _merge_A_Basc(*A, *B)


def _sc_topk_call(scores1, scores2, key_num):
    H, BT, KN = scores1.shape
    assert BT % NW == 0
    tpw = BT // NW
    NB = 8
    assert tpw % NB == 0
    n_chunks = KN // L
    nc_cand = 128 // L
    nbatches = tpw // NB

    mesh = plsc.VectorSubcoreMesh(core_axis_name="c", subcore_axis_name="s",
                                  num_cores=NC, num_subcores=NS)

    @functools.partial(
        pl.kernel,
        out_type=[jax.ShapeDtypeStruct((BT, H * KNN), jnp.float32),
                  jax.ShapeDtypeStruct((BT, H * KNN), jnp.int32)],
        mesh=mesh,
        compiler_params=pltpu.CompilerParams(needs_layout_passes=False),
        scratch_types=[
            pltpu.VMEM((2, H, NB, KN), jnp.float32),
            pltpu.VMEM((2, H, NB, KN), jnp.float32),
            pltpu.VMEM((4, 128), jnp.int32),
            pltpu.VMEM((4, 512), jnp.int32),
            pltpu.VMEM((4, 64), jnp.float32),
            pltpu.VMEM((4, 64), jnp.int32),
            pltpu.VMEM((2, KNN), jnp.float32),
            pltpu.VMEM((2, KNN), jnp.int32),
            pltpu.VMEM((2, KNN), jnp.float32),
            pltpu.VMEM((2, KNN), jnp.int32),
            pltpu.VMEM((NB, H * KNN), jnp.float32),
            pltpu.VMEM((NB, H * KNN), jnp.int32),
            pltpu.SemaphoreType.DMA,
        ],
    )
    def k(s1_hbm, s2_hbm, tab_hbm, outs_hbm, outi_hbm,
          b1, b2, tab, mk, sv, si, s1v, s1i, s2v, s2i, os_, oi_, sem):
        wid = lax.axis_index("c") * NS + lax.axis_index("s")
        tok0 = wid * tpw
        pltpu.sync_copy(tab_hbm, tab)
        neginf = jnp.float32(-jnp.inf)

        def copies(par, t0):
            for h in range(H):
                yield pltpu.make_async_copy(
                    s1_hbm.at[h, pl.ds(t0, NB), :], b1.at[par, h], sem)
                yield pltpu.make_async_copy(
                    s2_hbm.at[h, pl.ds(t0, NB), :], b2.at[par, h], sem)

        for c in copies(0, tok0):
            c.start()

        def process_row(par, h, rb, u):
            A0, IA0, A1, IA1 = _row_top32(b1, par, h, rb, n_chunks,
                                          mk, sv, si, 2 * u)
            B0, IB0, B1, IB1 = _row_top32(b2, par, h, rb, n_chunks,
                                          mk, sv, si, 2 * u + 1)
            s1v[u, pl.ds(0, L)] = A0
            s1v[u, pl.ds(L, L)] = A1
            s1i[u, pl.ds(0, L)] = IA0
            s1i[u, pl.ds(L, L)] = IA1
            s2v[u, pl.ds(0, L)] = B0
            s2v[u, pl.ds(L, L)] = B1
            s2i[u, pl.ds(0, L)] = IB0
            s2i[u, pl.ds(L, L)] = IB1

            def cand_chunk(cc):
                rj = tab[0, pl.ds(cc * L, L)]
                cj = tab[1, pl.ds(cc * L, L)]
                pj = tab[2, pl.ds(cc * L, L)]
                cv = (plsc.load_gather(s1v.at[u], [rj])
                      + plsc.load_gather(s2v.at[u], [cj]))
                cv = jnp.where(pj >= KNN * KNN, neginf, cv)
                return cv, pj

            cv0, pj0 = cand_chunk(0)
            cv1, pj1 = cand_chunk(1)
            st = _pair_init(cv0, pj0, cv1, pj1)
            for cc in range(2, nc_cand, 2):
                cva, pja = cand_chunk(cc)
                cvb, pjb = cand_chunk(cc + 1)
                st = _pair_step(*st, cva, pja, cvb, pjb)
            C0, P0, C1, P1 = st
            r0 = P0 >> 5
            c0 = P0 & (KNN - 1)
            r1 = P1 >> 5
            c1 = P1 & (KNN - 1)
            idx0 = (plsc.load_gather(s1i.at[u], [r0]) * key_num
                    + plsc.load_gather(s2i.at[u], [c0]))
            idx1 = (plsc.load_gather(s1i.at[u], [r1]) * key_num
                    + plsc.load_gather(s2i.at[u], [c1]))
            col = h * KNN
            os_[rb, pl.ds(col, L)] = C0
            os_[rb, pl.ds(col + L, L)] = C1
            oi_[rb, pl.ds(col, L)] = idx0
            oi_[rb, pl.ds(col + L, L)] = idx1

        def batch_body(bi, _):
            par = lax.rem(bi, 2)
            t0 = tok0 + bi * NB
            for c in copies(par, t0):
                c.wait()

            @pl.when(bi + 1 < nbatches)
            def _():
                for c in copies(1 - par, t0 + NB):
                    c.start()

            def rows_body(i, _):
                row = 2 * i
                h = row // NB
                rb = lax.rem(row, NB)
                process_row(par, h, rb, 0)
                process_row(par, h, rb + 1, 1)
                return 0

            lax.fori_loop(0, (H * NB) // 2, rows_body, 0)
            pltpu.sync_copy(os_, outs_hbm.at[pl.ds(t0, NB), :])
            pltpu.sync_copy(oi_, outi_hbm.at[pl.ds(t0, NB), :])
            return 0

        lax.fori_loop(0, nbatches, batch_body, 0)

    return k(scores1, scores2, jnp.asarray(_SC_CAND))


def kernel(x, conv_w, conv_b, Wq, bq, keys):
    B, T, C = x.shape
    QD = Wq.shape[0]
    half = keys.shape[1]
    key_dim = 2 * half
    heads = QD // key_dim
    key_num = keys.shape[0] // (2 * heads)
    BT = B * T

    xm1 = jnp.pad(x, ((0, 0), (1, 0), (0, 0)))[:, :T, :].reshape(BT, C)
    xm2 = jnp.pad(x, ((0, 0), (2, 0), (0, 0)))[:, :T, :].reshape(BT, C)
    xf = x.reshape(BT, C)
    cw = conv_w.T
    cb = conv_b[None, :]
    bq2 = bq[None, :]
    keysv = keys.reshape(heads, 2, key_num, half)
    keys1 = keysv[:, 0].reshape(heads * key_num, half)
    keys2 = keysv[:, 1].reshape(heads * key_num, half)

    TB = 256 if BT % 256 == 0 else BT
    grid = (BT // TB,)
    row_spec = pl.BlockSpec((TB, C), lambda i: (i, 0))
    full = lambda shape: pl.BlockSpec(shape, lambda i: tuple(0 for _ in shape))

    s1, s2 = pl.pallas_call(
        functools.partial(_scores_kernel, heads=heads, key_dim=key_dim,
                          key_num=key_num),
        grid=grid,
        in_specs=[
            row_spec, row_spec, row_spec,
            full(cw.shape), full(cb.shape), full(Wq.shape), full(bq2.shape),
            full(keys1.shape), full(keys2.shape),
        ],
        out_specs=[
            pl.BlockSpec((heads, TB, key_num), lambda i: (0, i, 0)),
            pl.BlockSpec((heads, TB, key_num), lambda i: (0, i, 0)),
        ],
        out_shape=[
            jax.ShapeDtypeStruct((heads, BT, key_num), jnp.float32),
            jax.ShapeDtypeStruct((heads, BT, key_num), jnp.float32),
        ],
    )(xf, xm1, xm2, cw, cb, Wq, bq2, keys1, keys2)

    scores, indices = _sc_topk_call(s1, s2, key_num)
    return (scores.reshape(B, T, heads * KNN),
            indices.reshape(B, T, heads * KNN))


# speedup vs baseline: 4.1565x; 4.1565x over previous
import functools

import jax
import jax.numpy as jnp
import numpy as np
from jax import lax
from jax.experimental import pallas as pl
from jax.experimental.pallas import tpu as pltpu
from jax.experimental.pallas import tpu_sc as plsc

KNN = 32
L = 16
NC, NS = 2, 16
NW = NC * NS


def _scores_kernel(xb, xm1, xm2, cw, cb, Wq, bq, keys1, keys2,
                   s1_out, s2_out, *, heads, key_dim, key_num):
    half = key_dim // 2
    conv = (cb[0][None, :]
            + xm2[...] * cw[0][None, :]
            + xm1[...] * cw[1][None, :]
            + xb[...] * cw[2][None, :])
    query = lax.dot_general(conv, Wq[...], (((1,), (1,)), ((), ())),
                            preferred_element_type=jnp.float32)
    query = query + bq[0][None, :]
    for h in range(heads):
        q1 = query[:, h * key_dim: h * key_dim + half]
        q2 = query[:, h * key_dim + half: (h + 1) * key_dim]
        k1 = keys1[h * key_num:(h + 1) * key_num, :]
        k2 = keys2[h * key_num:(h + 1) * key_num, :]
        s1_out[h, :, :] = lax.dot_general(
            q1, k1, (((1,), (1,)), ((), ())),
            preferred_element_type=jnp.float32)
        s2_out[h, :, :] = lax.dot_general(
            q2, k2, (((1,), (1,)), ((), ())),
            preferred_element_type=jnp.float32)


def _cand_table():
    pairs = [(r, c) for r in range(KNN) for c in range(KNN)
             if (r + 1) * (c + 1) <= KNN]
    pairs.sort(key=lambda rc: rc[0] * KNN + rc[1])
    npad = 128
    tab = np.zeros((4, npad), np.int32)
    tab[2, :] = 4 * KNN * KNN
    for j, (r, c) in enumerate(pairs):
        tab[0, j] = r
        tab[1, j] = c
        tab[2, j] = r * KNN + c
    return tab


_SC_CAND = _cand_table()


def _cmpsel(a, ia, b, ib):
    m = a >= b
    return (jnp.where(m, a, b), jnp.where(m, ia, ib),
            jnp.where(m, b, a), jnp.where(m, ib, ia))


def _sortkv(k, v, descending):
    return plsc.sort_key_val(k, v, descending=descending)


def _pair_init(c1, i1, c2, i2):
    b1k, b1v = _sortkv(c1, i1, False)
    b2k, b2v = _sortkv(c2, i2, True)
    hiB, ihiB, loB, iloB = _cmpsel(b1k, b1v, b2k, b2v)
    A0, IA0 = _sortkv(hiB, ihiB, True)
    A1, IA1 = _sortkv(loB, iloB, True)
    return A0, IA0, A1, IA1


def _pair_step(A0, IA0, A1, IA1, c1, i1, c2, i2):
    b1k, b1v = _sortkv(c1, i1, False)
    b2k, b2v = _sortkv(c2, i2, True)
    hiB, ihiB, loB, iloB = _cmpsel(b1k, b1v, b2k, b2v)
    Basc0, iB0 = _sortkv(loB, iloB, False)
    Basc1, iB1 = _sortkv(hiB, ihiB, False)
    h0, ih0, _, _ = _cmpsel(A0, IA0, Basc0, iB0)
    h1, ih1, _, _ = _cmpsel(A1, IA1, Basc1, iB1)
    u, iu, lo, ilo = _cmpsel(h0, ih0, h1, ih1)
    A0, IA0 = _sortkv(u, iu, True)
    A1, IA1 = _sortkv(lo, ilo, True)
    return A0, IA0, A1, IA1


def _pair_premerge_asc(c1, i1, c2, i2):
    b1k, b1v = _sortkv(c1, i1, False)
    b2k, b2v = _sortkv(c2, i2, True)
    hiB, ihiB, loB, iloB = _cmpsel(b1k, b1v, b2k, b2v)
    B0, iB0 = _sortkv(loB, iloB, False)
    B1, iB1 = _sortkv(hiB, ihiB, False)
    return B0, iB0, B1, iB1


def _merge_A_Basc(A0, IA0, A1, IA1, B0, iB0, B1, iB1):
    h0, ih0, _, _ = _cmpsel(A0, IA0, B0, iB0)
    h1, ih1, _, _ = _cmpsel(A1, IA1, B1, iB1)
    u, iu, lo, ilo = _cmpsel(h0, ih0, h1, ih1)
    A0, IA0 = _sortkv(u, iu, True)
    A1, IA1 = _sortkv(lo, ilo, True)
    return A0, IA0, A1, IA1


def _row_top32(buf, par, h, rb, n_chunks):
    iota = lax.broadcasted_iota(jnp.int32, (L,), 0)
    ld = lambda g: buf[par, h, rb, pl.ds(g * L, L)]
    st = _pair_init(ld(0), iota, ld(1), iota + L)
    pm = _pair_premerge_asc(ld(2), iota + 2 * L, ld(3), iota + 3 * L)
    for g in range(4, n_chunks, 2):
        pm_next = _pair_premerge_asc(ld(g), iota + g * L,
                                     ld(g + 1), iota + (g + 1) * L)
        st = _merge_A_Basc(*st, *pm)
        pm = pm_next
    return _merge_A_Basc(*st, *pm)


def _sc_topk_call(scores1, scores2, key_num):
    H, BT, KN = scores1.shape
    assert BT % NW == 0
    tpw = BT // NW
    NB = 8
    assert tpw % NB == 0
    n_chunks = KN // L
    nc_cand = 128 // L
    nbatches = tpw // NB

    mesh = plsc.VectorSubcoreMesh(core_axis_name="c", subcore_axis_name="s",
                                  num_cores=NC, num_subcores=NS)

    @functools.partial(
        pl.kernel,
        out_type=[jax.ShapeDtypeStruct((BT, H * KNN), jnp.float32),
                  jax.ShapeDtypeStruct((BT, H * KNN), jnp.int32)],
        mesh=mesh,
        compiler_params=pltpu.CompilerParams(needs_layout_passes=False),
        scratch_types=[
            pltpu.VMEM((2, H, NB, KN), jnp.float32),
            pltpu.VMEM((2, H, NB, KN), jnp.float32),
            pltpu.VMEM((4, 128), jnp.int32),
            pltpu.VMEM((2, KNN), jnp.float32),
            pltpu.VMEM((2, KNN), jnp.int32),
            pltpu.VMEM((2, KNN), jnp.float32),
            pltpu.VMEM((2, KNN), jnp.int32),
            pltpu.VMEM((NB, H * KNN), jnp.float32),
            pltpu.VMEM((NB, H * KNN), jnp.int32),
            pltpu.SemaphoreType.DMA,
        ],
    )
    def k(s1_hbm, s2_hbm, tab_hbm, outs_hbm, outi_hbm,
          b1, b2, tab, s1v, s1i, s2v, s2i, os_, oi_, sem):
        wid = lax.axis_index("c") * NS + lax.axis_index("s")
        tok0 = wid * tpw
        pltpu.sync_copy(tab_hbm, tab)
        neginf = jnp.float32(-jnp.inf)

        def copies(par, t0):
            for h in range(H):
                yield pltpu.make_async_copy(
                    s1_hbm.at[h, pl.ds(t0, NB), :], b1.at[par, h], sem)
                yield pltpu.make_async_copy(
                    s2_hbm.at[h, pl.ds(t0, NB), :], b2.at[par, h], sem)

        for c in copies(0, tok0):
            c.start()

        def process_row(par, h, rb, u):
            A0, IA0, A1, IA1 = _row_top32(b1, par, h, rb, n_chunks)
            B0, IB0, B1, IB1 = _row_top32(b2, par, h, rb, n_chunks)
            s1v[u, pl.ds(0, L)] = A0
            s1v[u, pl.ds(L, L)] = A1
            s1i[u, pl.ds(0, L)] = IA0
            s1i[u, pl.ds(L, L)] = IA1
            s2v[u, pl.ds(0, L)] = B0
            s2v[u, pl.ds(L, L)] = B1
            s2i[u, pl.ds(0, L)] = IB0
            s2i[u, pl.ds(L, L)] = IB1

            def cand_chunk(cc):
                rj = tab[0, pl.ds(cc * L, L)]
                cj = tab[1, pl.ds(cc * L, L)]
                pj = tab[2, pl.ds(cc * L, L)]
                cv = (plsc.load_gather(s1v.at[u], [rj])
                      + plsc.load_gather(s2v.at[u], [cj]))
                cv = jnp.where(pj >= KNN * KNN, neginf, cv)
                return cv, pj

            cv0, pj0 = cand_chunk(0)
            cv1, pj1 = cand_chunk(1)
            st = _pair_init(cv0, pj0, cv1, pj1)
            for cc in range(2, nc_cand, 2):
                cva, pja = cand_chunk(cc)
                cvb, pjb = cand_chunk(cc + 1)
                st = _pair_step(*st, cva, pja, cvb, pjb)
            C0, P0, C1, P1 = st
            r0 = P0 >> 5
            c0 = P0 & (KNN - 1)
            r1 = P1 >> 5
            c1 = P1 & (KNN - 1)
            idx0 = (plsc.load_gather(s1i.at[u], [r0]) * key_num
                    + plsc.load_gather(s2i.at[u], [c0]))
            idx1 = (plsc.load_gather(s1i.at[u], [r1]) * key_num
                    + plsc.load_gather(s2i.at[u], [c1]))
            col = h * KNN
            os_[rb, pl.ds(col, L)] = C0
            os_[rb, pl.ds(col + L, L)] = C1
            oi_[rb, pl.ds(col, L)] = idx0
            oi_[rb, pl.ds(col + L, L)] = idx1

        def batch_body(bi, _):
            par = lax.rem(bi, 2)
            t0 = tok0 + bi * NB
            for c in copies(par, t0):
                c.wait()

            @pl.when(bi + 1 < nbatches)
            def _():
                for c in copies(1 - par, t0 + NB):
                    c.start()

            def rows_body(i, _):
                row = 2 * i
                h = row // NB
                rb = lax.rem(row, NB)
                process_row(par, h, rb, 0)
                process_row(par, h, rb + 1, 1)
                return 0

            lax.fori_loop(0, (H * NB) // 2, rows_body, 0)
            pltpu.sync_copy(os_, outs_hbm.at[pl.ds(t0, NB), :])
            pltpu.sync_copy(oi_, outi_hbm.at[pl.ds(t0, NB), :])
            return 0

        lax.fori_loop(0, nbatches, batch_body, 0)

    return k(scores1, scores2, jnp.asarray(_SC_CAND))


def kernel(x, conv_w, conv_b, Wq, bq, keys):
    B, T, C = x.shape
    QD = Wq.shape[0]
    half = keys.shape[1]
    key_dim = 2 * half
    heads = QD // key_dim
    key_num = keys.shape[0] // (2 * heads)
    BT = B * T

    xm1 = jnp.pad(x, ((0, 0), (1, 0), (0, 0)))[:, :T, :].reshape(BT, C)
    xm2 = jnp.pad(x, ((0, 0), (2, 0), (0, 0)))[:, :T, :].reshape(BT, C)
    xf = x.reshape(BT, C)
    cw = conv_w.T
    cb = conv_b[None, :]
    bq2 = bq[None, :]
    keysv = keys.reshape(heads, 2, key_num, half)
    keys1 = keysv[:, 0].reshape(heads * key_num, half)
    keys2 = keysv[:, 1].reshape(heads * key_num, half)

    TB = 256 if BT % 256 == 0 else BT
    grid = (BT // TB,)
    row_spec = pl.BlockSpec((TB, C), lambda i: (i, 0))
    full = lambda shape: pl.BlockSpec(shape, lambda i: tuple(0 for _ in shape))

    s1, s2 = pl.pallas_call(
        functools.partial(_scores_kernel, heads=heads, key_dim=key_dim,
                          key_num=key_num),
        grid=grid,
        in_specs=[
            row_spec, row_spec, row_spec,
            full(cw.shape), full(cb.shape), full(Wq.shape), full(bq2.shape),
            full(keys1.shape), full(keys2.shape),
        ],
        out_specs=[
            pl.BlockSpec((heads, TB, key_num), lambda i: (0, i, 0)),
            pl.BlockSpec((heads, TB, key_num), lambda i: (0, i, 0)),
        ],
        out_shape=[
            jax.ShapeDtypeStruct((heads, BT, key_num), jnp.float32),
            jax.ShapeDtypeStruct((heads, BT, key_num), jnp.float32),
        ],
    )(xf, xm1, xm2, cw, cb, Wq, bq2, keys1, keys2)

    scores, indices = _sc_topk_call(s1, s2, key_num)
    return (scores.reshape(B, T, heads * KNN),
            indices.reshape(B, T, heads * KNN))
